# baseline (speedup 1.0000x reference)
import functools

import jax
import jax.numpy as jnp
from jax import lax
from jax.experimental import pallas as pl
from jax.experimental.pallas import tpu as pltpu
from jax.experimental.pallas import tpu_sc as plsc

HIDDEN = 64
PAIR = 2 * HIDDEN
NUM_WORKERS = 32
S_BLOCK = 8


SEG = (128, 72)


def _gather_pairs(pair_idx, packed):
    b, l = pair_idx.shape
    n = b * l
    rpw = b // NUM_WORKERS
    mesh = plsc.VectorSubcoreMesh(core_axis_name="c", subcore_axis_name="s")

    @functools.partial(
        pl.kernel,
        mesh=mesh,
        out_type=jax.ShapeDtypeStruct((n, PAIR), jnp.float32),
        scratch_types=[
            pltpu.VMEM((rpw, l), jnp.int32),
            pltpu.VMEM((SEG[0], PAIR), jnp.float32),
            pltpu.VMEM((SEG[1], PAIR), jnp.float32),
            pltpu.VMEM((SEG[0], PAIR), jnp.float32),
            pltpu.VMEM((SEG[1], PAIR), jnp.float32),
            pltpu.SemaphoreType.DMA,
            pltpu.SemaphoreType.DMA,
        ],
    )
    def k(idx_hbm, packed_hbm, out_hbm, idx_v, a0, b0_, a1, b1_, gsem, wsem):
        wid = lax.axis_index("s") * 2 + lax.axis_index("c")
        base = wid * rpw
        pltpu.sync_copy(idx_hbm.at[pl.ds(base, rpw)], idx_v)

        bufs = ((a0, b0_), (a1, b1_))
        gathers = [None] * rpw
        writes = [None] * rpw

        def start_gathers(g, bufpair):
            return (
                pltpu.async_copy(
                    packed_hbm.at[idx_v.at[g].at[pl.ds(0, SEG[0])]],
                    bufpair[0], gsem),
                pltpu.async_copy(
                    packed_hbm.at[idx_v.at[g].at[pl.ds(SEG[0], SEG[1])]],
                    bufpair[1], gsem),
            )

        gathers[0] = start_gathers(0, bufs[0])
        for g in range(rpw):
            for h in gathers[g]:
                h.wait()
            if g >= 1:
                for w in writes[g - 1]:
                    w.wait()
            if g + 1 < rpw:
                gathers[g + 1] = start_gathers(g + 1, bufs[(g + 1) % 2])
            row0 = (base + g) * l
            writes[g] = [
                pltpu.async_copy(
                    bufs[g % 2][0], out_hbm.at[pl.ds(row0, SEG[0])], wsem),
                pltpu.async_copy(
                    bufs[g % 2][1],
                    out_hbm.at[pl.ds(row0 + SEG[0], SEG[1])], wsem),
            ]
        for w in writes[rpw - 1]:
            w.wait()

    return k(pair_idx, packed)


def _select_half(pairs, x, b, l):
    def body(p_ref, x_ref, o_ref):
        p = p_ref[...].reshape(S_BLOCK, l, PAIR)
        odd = (x_ref[...] & 1)[:, :, None] == 1
        o_ref[...] = jnp.where(odd, p[:, :, HIDDEN:], p[:, :, :HIDDEN])

    return pl.pallas_call(
        body,
        grid=(b // S_BLOCK,),
        in_specs=[
            pl.BlockSpec((S_BLOCK * l, PAIR), lambda g: (g, 0)),
            pl.BlockSpec((S_BLOCK, l), lambda g: (g, 0)),
        ],
        out_specs=pl.BlockSpec((S_BLOCK, l, HIDDEN), lambda g: (g, 0, 0)),
        out_shape=jax.ShapeDtypeStruct((b, l, HIDDEN), jnp.float32),
    )(pairs, x)


def kernel(x, emb):
    b, l = x.shape
    if x.dtype != jnp.int32:
        x = x.astype(jnp.int32)
    packed = emb.reshape(emb.shape[0] // 2, PAIR)
    pairs = _gather_pairs(x >> 1, packed)
    return _select_half(pairs, x, b, l)


# speedup vs baseline: 1.1720x; 1.1720x over previous
import functools

import jax
import jax.numpy as jnp
from jax import lax
from jax.experimental import pallas as pl
from jax.experimental.pallas import tpu as pltpu
from jax.experimental.pallas import tpu_sc as plsc

HIDDEN = 64
NUM_WORKERS = 32
CHUNK = 800
SEG = (128, 72)


def _flatten_idx(x2d):
    b, l = x2d.shape
    n = b * l
    rpw = b // NUM_WORKERS
    mesh = plsc.VectorSubcoreMesh(core_axis_name="c", subcore_axis_name="s")

    lanes = 16
    offs = [c * lanes for c in range(l // lanes)] + [l - lanes]

    @functools.partial(
        pl.kernel,
        mesh=mesh,
        out_type=jax.ShapeDtypeStruct((n,), jnp.int32),
        scratch_types=[
            pltpu.VMEM((rpw, l), jnp.int32),
            pltpu.VMEM((rpw * l,), jnp.int32),
        ],
    )
    def k(x_hbm, flat_hbm, xt_v, f_v):
        wid = lax.axis_index("s") * 2 + lax.axis_index("c")
        base = wid * rpw
        pltpu.sync_copy(x_hbm.at[pl.ds(base, rpw)], xt_v)
        for r in range(rpw):
            for off in offs:
                f_v[pl.ds(r * l + off, lanes)] = xt_v[r, pl.ds(off, lanes)]
        pltpu.sync_copy(f_v, flat_hbm.at[pl.ds(base * l, rpw * l)])

    return k(x2d)


def _emb_lookup_sc(idx_flat, emb, b, l):
    n = b * l
    bpw = n // NUM_WORKERS
    nchunks = bpw // CHUNK
    rows_per_chunk = CHUNK // l
    mesh = plsc.VectorSubcoreMesh(core_axis_name="c", subcore_axis_name="s")

    @functools.partial(
        pl.kernel,
        mesh=mesh,
        out_type=jax.ShapeDtypeStruct((b, l, HIDDEN), jnp.float32),
        compiler_params=pltpu.CompilerParams(use_tc_tiling_on_sc=False),
        scratch_types=[
            pltpu.VMEM((bpw,), jnp.int32),
            pltpu.VMEM((CHUNK, HIDDEN), jnp.float32),
            pltpu.VMEM((CHUNK, HIDDEN), jnp.float32),
            pltpu.SemaphoreType.DMA,
            pltpu.SemaphoreType.DMA,
        ],
    )
    def k(idx_hbm, table_hbm, out_hbm, idx_v, buf0, buf1, gsem, wsem):
        wid = lax.axis_index("s") * 2 + lax.axis_index("c")
        base = wid * bpw
        pltpu.sync_copy(idx_hbm.at[pl.ds(base, bpw)], idx_v)

        bufs = (buf0, buf1)
        gathers = [None] * nchunks
        writes = [None] * nchunks

        gathers[0] = pltpu.async_copy(
            table_hbm.at[idx_v.at[pl.ds(0, CHUNK)]], bufs[0], gsem)
        for g in range(nchunks):
            gathers[g].wait()
            if g >= 1:
                for w in writes[g - 1]:
                    w.wait()
            if g + 1 < nchunks:
                gathers[g + 1] = pltpu.async_copy(
                    table_hbm.at[idx_v.at[pl.ds((g + 1) * CHUNK, CHUNK)]],
                    bufs[(g + 1) % 2], gsem)
            b0 = (base + g * CHUNK) // l
            writes[g] = [
                pltpu.async_copy(
                    bufs[g % 2].at[pl.ds(j * l, l)], out_hbm.at[b0 + j], wsem)
                for j in range(rows_per_chunk)
            ]
        for w in writes[nchunks - 1]:
            w.wait()

    return k(idx_flat, emb)


def kernel(x, emb):
    b, l = x.shape
    if x.dtype != jnp.int32:
        x = x.astype(jnp.int32)
    idx_flat = _flatten_idx(x)
    return _emb_lookup_sc(idx_flat, emb, b, l)
